# baseline (speedup 1.0000x reference)
import jax
import jax.numpy as jnp
from jax.experimental import pallas as pl
from jax.experimental.pallas import tpu as pltpu

_EPS = 1e-5


def _conv_stats_body(t_ref, w_ref, b_ref, o_ref, s_ref, q_ref):
    acc = jnp.dot(t_ref[...], w_ref[...], preferred_element_type=jnp.float32)
    y = jnp.maximum(acc + b_ref[...], 0.0)
    o_ref[...] = y.astype(jnp.bfloat16)
    s_ref[...] = jnp.sum(y, axis=0).reshape(1, 1, -1)
    q_ref[...] = jnp.sum(y * y, axis=0).reshape(1, 1, -1)


def _im2col_s2(x):
    n, h, w, cin = x.shape
    ho, wo = h // 2, w // 2
    xp = jnp.pad(x, ((0, 0), (1, 1), (1, 1), (0, 0)))
    slabs = []
    for kh in range(3):
        for kw in range(3):
            slabs.append(xp[:, kh:kh + 2 * ho - 1:2, kw:kw + 2 * wo - 1:2, :])
    pat = jnp.concatenate(slabs, axis=-1)
    return pat.reshape(n * ho * wo, 9 * cin), ho, wo


def _conv_bn_layer(x_bf, w, b, g, beta):
    n = x_bf.shape[0]
    taps, ho, wo = _im2col_s2(x_bf)
    m, k = taps.shape
    cout = w.shape[3]
    wk = w.reshape(k, cout).astype(jnp.bfloat16)
    tm = min(512, m)
    grid = m // tm

    out, ps, pq = pl.pallas_call(
        _conv_stats_body,
        out_shape=(
            jax.ShapeDtypeStruct((m, cout), jnp.bfloat16),
            jax.ShapeDtypeStruct((grid, 1, cout), jnp.float32),
            jax.ShapeDtypeStruct((grid, 1, cout), jnp.float32),
        ),
        grid=(grid,),
        in_specs=[
            pl.BlockSpec((tm, k), lambda i: (i, 0)),
            pl.BlockSpec((k, cout), lambda i: (0, 0)),
            pl.BlockSpec((1, cout), lambda i: (0, 0)),
        ],
        out_specs=(
            pl.BlockSpec((tm, cout), lambda i: (i, 0)),
            pl.BlockSpec((1, 1, cout), lambda i: (i, 0, 0)),
            pl.BlockSpec((1, 1, cout), lambda i: (i, 0, 0)),
        ),
        compiler_params=pltpu.CompilerParams(
            dimension_semantics=("parallel",),
            vmem_limit_bytes=100 * 1024 * 1024,
        ),
        cost_estimate=pl.CostEstimate(
            flops=2 * m * k * cout, transcendentals=0,
            bytes_accessed=m * k * 2 + k * cout * 2 + m * cout * 2),
    )(taps, wk, b.astype(jnp.float32).reshape(1, cout))

    mean = ps.sum(axis=0).reshape(cout) / m
    var = pq.sum(axis=0).reshape(cout) / m - mean * mean
    scale = g * jax.lax.rsqrt(var + _EPS)
    shift = beta - mean * scale
    xn = out * scale.astype(jnp.float32) + shift.astype(jnp.float32)
    return xn.astype(jnp.bfloat16).reshape(n, ho, wo, cout)


def _conv_planes_body(p00, p01, p10, p11, w_ref, b_ref, o_ref, s_ref, q_ref):
    _, ho, wo, cout = o_ref.shape
    cin = w_ref.shape[1]
    planes = ((p00, p01), (p10, p11))
    acc = None
    for kh in range(3):
        for kw in range(3):
            pr = planes[kh & 1][kw & 1]
            slab = pr[0, (kh >> 1):(kh >> 1) + ho,
                      (kw >> 1):(kw >> 1) + wo, :].reshape(ho * wo, cin)
            d = jnp.dot(slab, w_ref[kh * 3 + kw],
                        preferred_element_type=jnp.float32)
            acc = d if acc is None else acc + d
    y = jnp.maximum(acc + b_ref[...], 0.0)
    o_ref[...] = y.reshape(1, ho, wo, cout).astype(jnp.bfloat16)
    s_ref[...] = jnp.sum(y, axis=0).reshape(1, 1, -1)
    q_ref[...] = jnp.sum(y * y, axis=0).reshape(1, 1, -1)


def _conv_bn_layer_planes(x_bf, w, b, g, beta):
    n, h, wsp, cin = x_bf.shape
    ho, wo = h // 2, wsp // 2
    cout = w.shape[3]
    m = n * ho * wo

    xp = jnp.pad(x_bf, ((0, 0), (1, 1), (1, 1), (0, 0)))
    pe, po = xp[:, 0::2], xp[:, 1::2]
    quads = [pe[:, :, 0::2], pe[:, :, 1::2], po[:, :, 0::2], po[:, :, 1::2]]
    wq = w.reshape(9, cin, cout).astype(jnp.bfloat16)

    plane_spec = pl.BlockSpec((1, ho + 1, wo + 1, cin),
                              lambda i: (i, 0, 0, 0))
    out, ps, pq = pl.pallas_call(
        _conv_planes_body,
        out_shape=(
            jax.ShapeDtypeStruct((n, ho, wo, cout), jnp.bfloat16),
            jax.ShapeDtypeStruct((n, 1, cout), jnp.float32),
            jax.ShapeDtypeStruct((n, 1, cout), jnp.float32),
        ),
        grid=(n,),
        in_specs=[plane_spec] * 4 + [
            pl.BlockSpec((9, cin, cout), lambda i: (0, 0, 0)),
            pl.BlockSpec((1, cout), lambda i: (0, 0)),
        ],
        out_specs=(
            pl.BlockSpec((1, ho, wo, cout), lambda i: (i, 0, 0, 0)),
            pl.BlockSpec((1, 1, cout), lambda i: (i, 0, 0)),
            pl.BlockSpec((1, 1, cout), lambda i: (i, 0, 0)),
        ),
        compiler_params=pltpu.CompilerParams(
            dimension_semantics=("parallel",),
            vmem_limit_bytes=100 * 1024 * 1024,
        ),
        cost_estimate=pl.CostEstimate(
            flops=2 * m * 9 * cin * cout, transcendentals=0,
            bytes_accessed=n * (ho + 1) * (wo + 1) * cin * 2 * 4
            + 9 * cin * cout * 2 + m * cout * 2),
    )(*quads, wq, b.astype(jnp.float32).reshape(1, cout))

    mean = ps.sum(axis=0).reshape(cout) / m
    var = pq.sum(axis=0).reshape(cout) / m - mean * mean
    scale = g * jax.lax.rsqrt(var + _EPS)
    shift = beta - mean * scale
    xn = out * scale.astype(jnp.float32) + shift.astype(jnp.float32)
    return xn.astype(jnp.bfloat16)


def _axis_idx(out_size, in_size):
    sc = (in_size - 1) / (out_size - 1) if out_size > 1 else 0.0
    f = jnp.arange(out_size, dtype=jnp.float32) * sc
    lo = jnp.clip(jnp.floor(f).astype(jnp.int32), 0, in_size - 1)
    hi = jnp.minimum(lo + 1, in_size - 1)
    return lo, hi, f - lo.astype(jnp.float32)


def _resize_half(y, oh, ow):
    h0, h1, th = _axis_idx(oh, y.shape[1])
    w0, w1, tw = _axis_idx(ow, y.shape[2])
    th = th[None, :, None, None].astype(y.dtype)
    tw = tw[None, None, :, None].astype(y.dtype)
    r0, r1 = y[:, h0], y[:, h1]
    top = r0[:, :, w0] * (1 - tw) + r0[:, :, w1] * tw
    bot = r1[:, :, w0] * (1 - tw) + r1[:, :, w1] * tw
    return top * (1 - th) + bot * th


def _interp_mat(out_size, in_size):
    lo, hi, t = _axis_idx(out_size, in_size)
    cols = jnp.arange(in_size)[None, :]
    t = t[:, None]
    return ((cols == lo[:, None]) * (1.0 - t)
            + (cols == hi[:, None]) * t).astype(jnp.float32)


def _make_upsum_body(heights, cb, kp, out_w):
    def body(rh_ref, rw1, rw2, rw3, rw4, y1, y2, y3, y4, o_ref):
        parts = []
        for y_ref, rw_ref, h in ((y1, rw1, heights[0]), (y2, rw2, heights[1]),
                                 (y3, rw3, heights[2]), (y4, rw4, heights[3])):
            w = y_ref.shape[3]
            u = jnp.dot(y_ref[...].reshape(cb * h, w), rw_ref[...],
                        preferred_element_type=jnp.float32)
            parts.append(u.astype(jnp.bfloat16).reshape(cb, h, out_w))
        pad = kp - sum(heights)
        if pad:
            parts.append(jnp.zeros((cb, pad, out_w), jnp.bfloat16))
        ucat = jnp.concatenate(parts, axis=1)
        rh = rh_ref[...]
        for c in range(cb):
            o_ref[0, c] = jnp.dot(rh, ucat[c],
                                  preferred_element_type=jnp.float32)
    return body


def _upsample_sum(ys, out_h, out_w):
    c, n = ys[0].shape[0], ys[0].shape[1]
    heights = tuple(y.shape[2] for y in ys)
    kp = ((sum(heights) + 127) // 128) * 128

    rh = jnp.concatenate([_interp_mat(out_h, h) for h in heights], axis=1)
    rh = jnp.pad(rh, ((0, 0), (0, kp - sum(heights)))).astype(jnp.bfloat16)
    rws = [_interp_mat(out_w, y.shape[3]).T.astype(jnp.bfloat16) for y in ys]

    cb = 32
    grid = (n, c // cb)

    y_specs = [
        pl.BlockSpec((cb, 1) + y.shape[2:], lambda i, j: (j, i, 0, 0))
        for y in ys
    ]
    rw_specs = [pl.BlockSpec(r.shape, lambda i, j: (0, 0)) for r in rws]

    out = pl.pallas_call(
        _make_upsum_body(heights, cb, kp, out_w),
        out_shape=jax.ShapeDtypeStruct((n, c, out_h, out_w), jnp.float32),
        grid=grid,
        in_specs=[pl.BlockSpec((out_h, kp), lambda i, j: (0, 0))]
        + rw_specs + y_specs,
        out_specs=pl.BlockSpec((1, cb, out_h, out_w),
                               lambda i, j: (i, j, 0, 0)),
        compiler_params=pltpu.CompilerParams(
            dimension_semantics=("parallel", "parallel"),
            vmem_limit_bytes=100 * 1024 * 1024,
        ),
        cost_estimate=pl.CostEstimate(
            flops=2 * n * c * out_h * kp * out_w,
            transcendentals=0,
            bytes_accessed=sum(y.size for y in ys) * 2
            + n * c * out_h * out_w * 4),
    )(rh, *rws, *ys)
    return out


def kernel(x, w1, b1, g1, beta1, w2, b2, g2, beta2,
           w3, b3, g3, beta3, w4, b4, g4, beta4):
    xh = jnp.transpose(x, (0, 2, 3, 1)).astype(jnp.bfloat16)
    h, w = xh.shape[1], xh.shape[2]

    x1 = _conv_bn_layer(xh, w1, b1, g1, beta1)
    y1 = x1
    x2 = _conv_bn_layer_planes(x1, w2, b2, g2, beta2)
    y2 = _resize_half(y1, x2.shape[1], x2.shape[2]) + x2
    x3 = _conv_bn_layer_planes(x2, w3, b3, g3, beta3)
    y3 = _resize_half(y2, x3.shape[1], x3.shape[2]) + x3
    x4 = _conv_bn_layer_planes(x3, w4, b4, g4, beta4)
    y4 = _resize_half(y3, x4.shape[1], x4.shape[2]) + x4

    yts = [jnp.transpose(y, (3, 0, 1, 2)) for y in (y1, y2, y3, y4)]
    return _upsample_sum(yts, h, w)


# speedup vs baseline: 1.2136x; 1.2136x over previous
import jax
import jax.numpy as jnp
from jax.experimental import pallas as pl
from jax.experimental.pallas import tpu as pltpu

_EPS = 1e-5


def _conv_stats_body(t_ref, w_ref, b_ref, o_ref, s_ref, q_ref):
    acc = jnp.dot(t_ref[...], w_ref[...], preferred_element_type=jnp.float32)
    y = jnp.maximum(acc + b_ref[...], 0.0)
    o_ref[...] = y.astype(jnp.bfloat16)
    s_ref[...] = jnp.sum(y, axis=0).reshape(1, 1, -1)
    q_ref[...] = jnp.sum(y * y, axis=0).reshape(1, 1, -1)


def _im2col_s2(x):
    n, h, w, cin = x.shape
    ho, wo = h // 2, w // 2
    xp = jnp.pad(x, ((0, 0), (1, 1), (1, 1), (0, 0)))
    slabs = []
    for kh in range(3):
        for kw in range(3):
            slabs.append(xp[:, kh:kh + 2 * ho - 1:2, kw:kw + 2 * wo - 1:2, :])
    pat = jnp.concatenate(slabs, axis=-1)
    return pat.reshape(n * ho * wo, 9 * cin), ho, wo


def _conv_bn_layer(x_bf, w, b, g, beta):
    n = x_bf.shape[0]
    taps, ho, wo = _im2col_s2(x_bf)
    m, k = taps.shape
    cout = w.shape[3]
    wk = w.reshape(k, cout).astype(jnp.bfloat16)
    tm = min(512, m)
    grid = m // tm

    out, ps, pq = pl.pallas_call(
        _conv_stats_body,
        out_shape=(
            jax.ShapeDtypeStruct((m, cout), jnp.bfloat16),
            jax.ShapeDtypeStruct((grid, 1, cout), jnp.float32),
            jax.ShapeDtypeStruct((grid, 1, cout), jnp.float32),
        ),
        grid=(grid,),
        in_specs=[
            pl.BlockSpec((tm, k), lambda i: (i, 0)),
            pl.BlockSpec((k, cout), lambda i: (0, 0)),
            pl.BlockSpec((1, cout), lambda i: (0, 0)),
        ],
        out_specs=(
            pl.BlockSpec((tm, cout), lambda i: (i, 0)),
            pl.BlockSpec((1, 1, cout), lambda i: (i, 0, 0)),
            pl.BlockSpec((1, 1, cout), lambda i: (i, 0, 0)),
        ),
        compiler_params=pltpu.CompilerParams(
            dimension_semantics=("parallel",),
            vmem_limit_bytes=100 * 1024 * 1024,
        ),
        cost_estimate=pl.CostEstimate(
            flops=2 * m * k * cout, transcendentals=0,
            bytes_accessed=m * k * 2 + k * cout * 2 + m * cout * 2),
    )(taps, wk, b.astype(jnp.float32).reshape(1, cout))

    mean = ps.sum(axis=0).reshape(cout) / m
    var = pq.sum(axis=0).reshape(cout) / m - mean * mean
    scale = g * jax.lax.rsqrt(var + _EPS)
    shift = beta - mean * scale
    xn = out * scale.astype(jnp.float32) + shift.astype(jnp.float32)
    return xn.astype(jnp.bfloat16).reshape(n, ho, wo, cout)


def _axis_idx(out_size, in_size):
    sc = (in_size - 1) / (out_size - 1) if out_size > 1 else 0.0
    f = jnp.arange(out_size, dtype=jnp.float32) * sc
    lo = jnp.clip(jnp.floor(f).astype(jnp.int32), 0, in_size - 1)
    hi = jnp.minimum(lo + 1, in_size - 1)
    return lo, hi, f - lo.astype(jnp.float32)


def _resize_half(y, oh, ow):
    h0, h1, th = _axis_idx(oh, y.shape[1])
    w0, w1, tw = _axis_idx(ow, y.shape[2])
    th = th[None, :, None, None].astype(y.dtype)
    tw = tw[None, None, :, None].astype(y.dtype)
    r0, r1 = y[:, h0], y[:, h1]
    top = r0[:, :, w0] * (1 - tw) + r0[:, :, w1] * tw
    bot = r1[:, :, w0] * (1 - tw) + r1[:, :, w1] * tw
    return top * (1 - th) + bot * th


def _interp_mat(out_size, in_size):
    lo, hi, t = _axis_idx(out_size, in_size)
    cols = jnp.arange(in_size)[None, :]
    t = t[:, None]
    return ((cols == lo[:, None]) * (1.0 - t)
            + (cols == hi[:, None]) * t).astype(jnp.float32)


def _make_upsum_body(heights, cb, kp, out_w):
    def body(rh_ref, rw1, rw2, rw3, rw4, y1, y2, y3, y4, o_ref):
        parts = []
        for y_ref, rw_ref, h in ((y1, rw1, heights[0]), (y2, rw2, heights[1]),
                                 (y3, rw3, heights[2]), (y4, rw4, heights[3])):
            w = y_ref.shape[3]
            u = jnp.dot(y_ref[...].reshape(cb * h, w), rw_ref[...],
                        preferred_element_type=jnp.float32)
            parts.append(u.astype(jnp.bfloat16).reshape(cb, h, out_w))
        pad = kp - sum(heights)
        if pad:
            parts.append(jnp.zeros((cb, pad, out_w), jnp.bfloat16))
        ucat = jnp.concatenate(parts, axis=1)
        rh = rh_ref[...]
        for c in range(cb):
            o_ref[0, c] = jnp.dot(rh, ucat[c],
                                  preferred_element_type=jnp.float32)
    return body


def _upsample_sum(ys, out_h, out_w):
    c, n = ys[0].shape[0], ys[0].shape[1]
    heights = tuple(y.shape[2] for y in ys)
    kp = ((sum(heights) + 127) // 128) * 128

    rh = jnp.concatenate([_interp_mat(out_h, h) for h in heights], axis=1)
    rh = jnp.pad(rh, ((0, 0), (0, kp - sum(heights)))).astype(jnp.bfloat16)
    rws = [_interp_mat(out_w, y.shape[3]).T.astype(jnp.bfloat16) for y in ys]

    cb = 32
    grid = (n, c // cb)

    y_specs = [
        pl.BlockSpec((cb, 1) + y.shape[2:], lambda i, j: (j, i, 0, 0))
        for y in ys
    ]
    rw_specs = [pl.BlockSpec(r.shape, lambda i, j: (0, 0)) for r in rws]

    out = pl.pallas_call(
        _make_upsum_body(heights, cb, kp, out_w),
        out_shape=jax.ShapeDtypeStruct((n, c, out_h, out_w), jnp.float32),
        grid=grid,
        in_specs=[pl.BlockSpec((out_h, kp), lambda i, j: (0, 0))]
        + rw_specs + y_specs,
        out_specs=pl.BlockSpec((1, cb, out_h, out_w),
                               lambda i, j: (i, j, 0, 0)),
        compiler_params=pltpu.CompilerParams(
            dimension_semantics=("parallel", "parallel"),
            vmem_limit_bytes=100 * 1024 * 1024,
        ),
        cost_estimate=pl.CostEstimate(
            flops=2 * n * c * out_h * kp * out_w,
            transcendentals=0,
            bytes_accessed=sum(y.size for y in ys) * 2
            + n * c * out_h * out_w * 4),
    )(rh, *rws, *ys)
    return out


def kernel(x, w1, b1, g1, beta1, w2, b2, g2, beta2,
           w3, b3, g3, beta3, w4, b4, g4, beta4):
    xh = jnp.transpose(x, (0, 2, 3, 1)).astype(jnp.bfloat16)
    h, w = xh.shape[1], xh.shape[2]

    x1 = _conv_bn_layer(xh, w1, b1, g1, beta1)
    y1 = x1
    x2 = _conv_bn_layer(x1, w2, b2, g2, beta2)
    y2 = _resize_half(y1, x2.shape[1], x2.shape[2]) + x2
    x3 = _conv_bn_layer(x2, w3, b3, g3, beta3)
    y3 = _resize_half(y2, x3.shape[1], x3.shape[2]) + x3
    x4 = _conv_bn_layer(x3, w4, b4, g4, beta4)
    y4 = _resize_half(y3, x4.shape[1], x4.shape[2]) + x4

    yts = [jnp.transpose(y, (3, 0, 1, 2)) for y in (y1, y2, y3, y4)]
    return _upsample_sum(yts, h, w)
